# baseline (speedup 1.0000x reference)
import functools

import jax
import jax.numpy as jnp
from jax.experimental import pallas as pl
from jax.experimental.pallas import tpu as pltpu

NQ = 4
K = 1024
D = 256
B = 4096
BM = 128

_F32 = jnp.float32
_BF = jnp.bfloat16


def _split3(x):
    hi = x.astype(_BF)
    r1 = x - hi.astype(_F32)
    mid = r1.astype(_BF)
    lo = (r1 - mid.astype(_F32)).astype(_BF)
    return hi, mid, lo


def _dot(a, b):
    return jax.lax.dot_general(
        a, b, (((1,), (0,)), ((), ())), preferred_element_type=_F32
    )


def _rq_body(
    x_ref, cb_ref, cbT_ref, qout_ref, idx_ref,
    res_ref, tn_ref, th_ref, tm_ref, tl_ref, gh_ref, gm_ref, gl_ref,
):
    b = pl.program_id(0)
    i = pl.program_id(1)
    iota_k = jax.lax.broadcasted_iota(jnp.int32, (BM, K), 1)
    iota_q = jax.lax.broadcasted_iota(jnp.int32, (BM, NQ), 1)

    @pl.when(b == 0)
    def _prep_stage_tables():
        cb = cb_ref[i]
        cbT = cbT_ref[i]
        h, m, l = _split3(cbT)
        th_ref[i], tm_ref[i], tl_ref[i] = h, m, l
        h2, m2_, l2 = _split3(cb)
        gh_ref[i], gm_ref[i], gl_ref[i] = h2, m2_, l2
        tn_ref[i] = jnp.sum(cbT * cbT, axis=0, keepdims=True)

    @pl.when((b > 0) & (i == 0))
    def _init():
        res_ref[...] = x_ref[...]
        qout_ref[...] = jnp.zeros((BM, D), _F32)
        idx_ref[...] = jnp.zeros((BM, NQ), jnp.int32)

    @pl.when(b > 0)
    def _compute_stage():
        r = res_ref[...]
        r_hi = r.astype(_BF)
        r_lo = (r - r_hi.astype(_F32)).astype(_BF)
        cnorm = tn_ref[i]
        rc = _dot(r_hi, th_ref[i]) + (_dot(r_hi, tm_ref[i]) + _dot(r_lo, th_ref[i]))
        scores = cnorm - 2.0 * rc
        m1 = jnp.min(scores, axis=1, keepdims=True)
        i1 = jnp.min(jnp.where(scores == m1, iota_k, K), axis=1)
        masked = jnp.where(iota_k == i1[:, None], jnp.inf, scores)
        m2 = jnp.min(masked, axis=1, keepdims=True)
        i2 = jnp.min(jnp.where(masked == m2, iota_k, K), axis=1)
        oh1 = (iota_k == i1[:, None]).astype(_BF)
        oh2 = (iota_k == i2[:, None]).astype(_BF)
        q1 = (_dot(oh1, gh_ref[i]) + _dot(oh1, gm_ref[i])) + _dot(oh1, gl_ref[i])
        q2 = (_dot(oh2, gh_ref[i]) + _dot(oh2, gm_ref[i])) + _dot(oh2, gl_ref[i])
        e1 = r - q1
        e2 = r - q2
        d1 = jnp.sum(e1 * e1, axis=1)
        d2 = jnp.sum(e2 * e2, axis=1)
        take2 = (d2 < d1) | ((d2 == d1) & (i2 < i1))
        q = jnp.where(take2[:, None], q2, q1)
        idx = jnp.where(take2, i2, i1).astype(jnp.int32)
        res_ref[...] = r - q
        qout_ref[...] = qout_ref[...] + q
        idx_ref[...] = jnp.where(iota_q == i, idx[:, None], idx_ref[...])


@functools.partial(jax.jit, static_argnames=())
def kernel(inputs, codebook):
    codebook_t = jnp.transpose(codebook, (0, 2, 1))
    qout, idx = pl.pallas_call(
        _rq_body,
        grid=(B // BM + 1, NQ),
        in_specs=[
            pl.BlockSpec((BM, D), lambda b, i: (jnp.maximum(b - 1, 0), 0)),
            pl.BlockSpec((NQ, K, D), lambda b, i: (0, 0, 0)),
            pl.BlockSpec((NQ, D, K), lambda b, i: (0, 0, 0)),
        ],
        out_specs=[
            pl.BlockSpec((BM, D), lambda b, i: (jnp.maximum(b - 1, 0), 0)),
            pl.BlockSpec((BM, NQ), lambda b, i: (jnp.maximum(b - 1, 0), 0)),
        ],
        out_shape=[
            jax.ShapeDtypeStruct((B, D), jnp.float32),
            jax.ShapeDtypeStruct((B, NQ), jnp.int32),
        ],
        scratch_shapes=[
            pltpu.VMEM((BM, D), _F32),
            pltpu.VMEM((NQ, 1, K), _F32),
            pltpu.VMEM((NQ, D, K), _BF),
            pltpu.VMEM((NQ, D, K), _BF),
            pltpu.VMEM((NQ, D, K), _BF),
            pltpu.VMEM((NQ, K, D), _BF),
            pltpu.VMEM((NQ, K, D), _BF),
            pltpu.VMEM((NQ, K, D), _BF),
        ],
    )(inputs, codebook, codebook_t)
    return qout, idx


# speedup vs baseline: 1.6912x; 1.6912x over previous
import functools

import jax
import jax.numpy as jnp
from jax.experimental import pallas as pl
from jax.experimental.pallas import tpu as pltpu

NQ = 4
K = 1024
D = 256
B = 4096
BM = 1024

_F32 = jnp.float32
_BF = jnp.bfloat16


def _split3(x):
    hi = x.astype(_BF)
    r1 = x - hi.astype(_F32)
    mid = r1.astype(_BF)
    lo = (r1 - mid.astype(_F32)).astype(_BF)
    return hi, mid, lo


def _dot(a, b):
    return jax.lax.dot_general(
        a, b, (((1,), (0,)), ((), ())), preferred_element_type=_F32
    )


def _rq_body(
    x_ref, cb_ref, cbT_ref, qout_ref, idx_ref,
    res_ref, tn_ref, th_ref, tm_ref, tl_ref, gh_ref, gm_ref, gl_ref,
):
    b = pl.program_id(0)
    i = pl.program_id(1)
    iota_k = jax.lax.broadcasted_iota(jnp.int32, (BM, K), 1)
    iota_q = jax.lax.broadcasted_iota(jnp.int32, (BM, NQ), 1)

    @pl.when(b == 0)
    def _prep_stage_tables():
        cb = cb_ref[i]
        cbT = cbT_ref[i]
        h, m, l = _split3(cbT)
        th_ref[i], tm_ref[i], tl_ref[i] = h, m, l
        h2, m2_, l2 = _split3(cb)
        gh_ref[i], gm_ref[i], gl_ref[i] = h2, m2_, l2
        tn_ref[i] = jnp.sum(cbT * cbT, axis=0, keepdims=True)

    @pl.when((b > 0) & (i == 0))
    def _init():
        res_ref[...] = x_ref[...]
        qout_ref[...] = jnp.zeros((BM, D), _F32)
        idx_ref[...] = jnp.zeros((BM, NQ), jnp.int32)

    @pl.when(b > 0)
    def _compute_stage():
        r = res_ref[...]
        r_hi = r.astype(_BF)
        r_lo = (r - r_hi.astype(_F32)).astype(_BF)
        cnorm = tn_ref[i]
        rc = _dot(r_hi, th_ref[i]) + (_dot(r_hi, tm_ref[i]) + _dot(r_lo, th_ref[i]))
        scores = cnorm - 2.0 * rc
        m1 = jnp.min(scores, axis=1, keepdims=True)
        i1 = jnp.min(jnp.where(scores == m1, iota_k, K), axis=1)
        masked = jnp.where(iota_k == i1[:, None], jnp.inf, scores)
        m2 = jnp.min(masked, axis=1, keepdims=True)
        i2 = jnp.min(jnp.where(masked == m2, iota_k, K), axis=1)
        oh1 = (iota_k == i1[:, None]).astype(_BF)
        oh2 = (iota_k == i2[:, None]).astype(_BF)
        q1 = (_dot(oh1, gh_ref[i]) + _dot(oh1, gm_ref[i])) + _dot(oh1, gl_ref[i])
        q2 = (_dot(oh2, gh_ref[i]) + _dot(oh2, gm_ref[i])) + _dot(oh2, gl_ref[i])
        e1 = r - q1
        e2 = r - q2
        d1 = jnp.sum(e1 * e1, axis=1)
        d2 = jnp.sum(e2 * e2, axis=1)
        take2 = (d2 < d1) | ((d2 == d1) & (i2 < i1))
        q = jnp.where(take2[:, None], q2, q1)
        idx = jnp.where(take2, i2, i1).astype(jnp.int32)
        res_ref[...] = r - q
        qout_ref[...] = qout_ref[...] + q
        idx_ref[...] = jnp.where(iota_q == i, idx[:, None], idx_ref[...])


@functools.partial(jax.jit, static_argnames=())
def kernel(inputs, codebook):
    codebook_t = jnp.transpose(codebook, (0, 2, 1))
    qout, idx = pl.pallas_call(
        _rq_body,
        grid=(B // BM + 1, NQ),
        in_specs=[
            pl.BlockSpec((BM, D), lambda b, i: (jnp.maximum(b - 1, 0), 0)),
            pl.BlockSpec((NQ, K, D), lambda b, i: (0, 0, 0)),
            pl.BlockSpec((NQ, D, K), lambda b, i: (0, 0, 0)),
        ],
        out_specs=[
            pl.BlockSpec((BM, D), lambda b, i: (jnp.maximum(b - 1, 0), 0)),
            pl.BlockSpec((BM, NQ), lambda b, i: (jnp.maximum(b - 1, 0), 0)),
        ],
        out_shape=[
            jax.ShapeDtypeStruct((B, D), jnp.float32),
            jax.ShapeDtypeStruct((B, NQ), jnp.int32),
        ],
        scratch_shapes=[
            pltpu.VMEM((BM, D), _F32),
            pltpu.VMEM((NQ, 1, K), _F32),
            pltpu.VMEM((NQ, D, K), _BF),
            pltpu.VMEM((NQ, D, K), _BF),
            pltpu.VMEM((NQ, D, K), _BF),
            pltpu.VMEM((NQ, K, D), _BF),
            pltpu.VMEM((NQ, K, D), _BF),
            pltpu.VMEM((NQ, K, D), _BF),
        ],
    )(inputs, codebook, codebook_t)
    return qout, idx
